# baseline (speedup 1.0000x reference)
import jax
import jax.numpy as jnp
from jax.experimental import pallas as pl
from jax.experimental.pallas import tpu as pltpu

_BB = 4
_PB = 512


def _add_kernel(x_ref, pos_ref, o_ref):
    j = pl.program_id(1)
    pos = pos_ref[pl.ds(j * _PB, _PB), :]
    o_ref[...] = x_ref[...] + pos[None, :, :]


def kernel(input_patch, pos_table):
    B, P, D = input_patch.shape
    return pl.pallas_call(
        _add_kernel,
        grid=(B // _BB, P // _PB),
        in_specs=[
            pl.BlockSpec((_BB, _PB, D), lambda i, j: (i, j, 0)),
            pl.BlockSpec((P, D), lambda i, j: (0, 0)),
        ],
        out_specs=pl.BlockSpec((_BB, _PB, D), lambda i, j: (i, j, 0)),
        out_shape=jax.ShapeDtypeStruct((B, P, D), input_patch.dtype),
    )(input_patch, pos_table)


# speedup vs baseline: 1.0097x; 1.0097x over previous
import jax
import jax.numpy as jnp
from jax.experimental import pallas as pl

_BB = 4


def _add_kernel(x_ref, pos_ref, o_ref):
    o_ref[...] = x_ref[...] + pos_ref[...][None, :, :]


def kernel(input_patch, pos_table):
    B, P, D = input_patch.shape
    grid = (B // _BB,)
    return pl.pallas_call(
        _add_kernel,
        grid=grid,
        in_specs=[
            pl.BlockSpec((_BB, P, D), lambda i: (i, 0, 0)),
            pl.BlockSpec((P, D), lambda i: (0, 0)),
        ],
        out_specs=pl.BlockSpec((_BB, P, D), lambda i: (i, 0, 0)),
        out_shape=jax.ShapeDtypeStruct((B, P, D), input_patch.dtype),
    )(input_patch, pos_table)
